# baseline (speedup 1.0000x reference)
import functools

import jax
import jax.numpy as jnp
from jax import lax
from jax.experimental import pallas as pl
from jax.experimental.pallas import tpu as pltpu
from jax.experimental.pallas import tpu_sc as plsc

N = 10000
D = 128
G = 100
P = 100
LANES = 16
NTILES = 16
NCORES = 2
NS = 10240
TRASH = N
B = 128
ROWS_PER_TILE = NS // NTILES
ZCHUNKS = ROWS_PER_TILE // LANES
WCHUNKS = ROWS_PER_TILE // B
GRPS = B // LANES
DC = D // LANES


def _proj_body(x_ref, w_ref, asrc_ref, adst_ref, h_ref, as_ref, ad_ref):
    h = jnp.dot(x_ref[...], w_ref[...], preferred_element_type=jnp.float32)
    h_ref[...] = h
    as_ref[...] = jnp.sum(h * asrc_ref[...], axis=1, keepdims=True)
    ad_ref[...] = jnp.sum(h * adst_ref[...], axis=1, keepdims=True)


def _project(x, w, a_src, a_dst):
    n = x.shape[0]
    blk = 2000
    grid = n // blk
    return pl.pallas_call(
        _proj_body,
        grid=(grid,),
        in_specs=[
            pl.BlockSpec((blk, D), lambda i: (i, 0)),
            pl.BlockSpec((D, D), lambda i: (0, 0)),
            pl.BlockSpec((1, D), lambda i: (0, 0)),
            pl.BlockSpec((1, D), lambda i: (0, 0)),
        ],
        out_specs=[
            pl.BlockSpec((blk, D), lambda i: (i, 0)),
            pl.BlockSpec((blk, 1), lambda i: (i, 0)),
            pl.BlockSpec((blk, 1), lambda i: (i, 0)),
        ],
        out_shape=[
            jax.ShapeDtypeStruct((n, D), jnp.float32),
            jax.ShapeDtypeStruct((n, 1), jnp.float32),
            jax.ShapeDtypeStruct((n, 1), jnp.float32),
        ],
    )(x, w, a_src, a_dst)


def _group_body(x2_ref, w2_ref, a2s_ref, a2d_ref, b2_ref, ei2_ref,
                xg_ref, am_ref):
    h2 = jnp.dot(x2_ref[...], w2_ref[...], preferred_element_type=jnp.float32)
    as2 = jnp.sum(h2 * a2s_ref[...], axis=1)
    ad2 = jnp.sum(h2 * a2d_ref[...], axis=1)
    ei = ei2_ref[...]
    eg = ei.shape[1]
    gi = lax.broadcasted_iota(jnp.int32, (eg, G), 1)
    src_oh = (ei[0][:, None] == gi).astype(jnp.float32)
    dst_oh = (ei[1][:, None] == gi).astype(jnp.float32)
    m_cnt = lax.dot_general(src_oh, dst_oh, (((0,), (0,)), ((), ())),
                            preferred_element_type=jnp.float32)
    eye = (lax.broadcasted_iota(jnp.int32, (G, G), 0)
           == lax.broadcasted_iota(jnp.int32, (G, G), 1)).astype(jnp.float32)
    mp = m_cnt + eye
    emat = as2[:, None] + ad2[None, :]
    el = jnp.maximum(emat, 0.2 * emat)
    mx = jnp.max(jnp.where(mp > 0, el, -1e30), axis=0)
    z = mp * jnp.exp(el - mx[None, :])
    ssum = jnp.sum(z, axis=0)
    num = lax.dot_general(z, h2, (((0,), (0,)), ((), ())),
                          preferred_element_type=jnp.float32)
    xg = num / (ssum[:, None] + 1e-16) + b2_ref[...][None, :]
    xg_ref[...] = xg
    cnt = jnp.sum(m_cnt, axis=1)
    adj_sum = jnp.dot(m_cnt, xg, preferred_element_type=jnp.float32)
    am_ref[...] = adj_sum / jnp.maximum(cnt, 1.0)[:, None]


def _group_gat(x2, w2, a2_src, a2_dst, b2, ei2):
    return pl.pallas_call(
        _group_body,
        out_shape=[
            jax.ShapeDtypeStruct((G, D), jnp.float32),
            jax.ShapeDtypeStruct((G, D), jnp.float32),
        ],
    )(x2, w2, a2_src, a2_dst, b2, ei2)


def _load_node_arrays(as_hbm, ad_hbm, as_loc, ad_loc):
    pltpu.sync_copy(as_hbm, as_loc.at[pl.ds(0, N)])
    pltpu.sync_copy(ad_hbm, ad_loc.at[pl.ds(0, N)])
    for i in range((NS - N) // LANES):
        as_loc[pl.ds(N + i * LANES, LANES)] = jnp.zeros((LANES,), jnp.float32)
        ad_loc[pl.ds(N + i * LANES, LANES)] = jnp.zeros((LANES,), jnp.float32)


def _make_sc_pass1(chunks):
    mesh = plsc.VectorSubcoreMesh(core_axis_name="c", subcore_axis_name="s")

    @functools.partial(
        pl.kernel,
        out_type=jax.ShapeDtypeStruct((NCORES, NS), jnp.float32),
        mesh=mesh,
        compiler_params=pltpu.CompilerParams(needs_layout_passes=False),
        scratch_types=[
            pltpu.VMEM((NS,), jnp.float32),
            pltpu.VMEM((NS,), jnp.float32),
            pltpu.VMEM((1, chunks, B), jnp.int32),
            pltpu.VMEM((1, chunks, B), jnp.int32),
            pltpu.VMEM((B,), jnp.float32),
            pltpu.VMEM((B,), jnp.float32),
            pltpu.SemaphoreType.DMA,
            pltpu.SemaphoreType.DMA,
            pltpu.VMEM((ROWS_PER_TILE,), jnp.float32),
            pltpu.VMEM_SHARED((NS,), jnp.float32),
        ],
    )
    def pass1(src_hbm, dst_hbm, as_hbm, ad_hbm, s_out,
              as_loc, ad_loc, src2d, dst2d, ex_b0, ex_b1, sem0, sem1,
              zbuf, s_sh):
        c = lax.axis_index("c")
        t = lax.axis_index("s")
        wid = c * NTILES + t

        def zi(i, _):
            zbuf[pl.ds(i * LANES, LANES)] = jnp.zeros((LANES,), jnp.float32)
            return 0
        lax.fori_loop(0, ZCHUNKS, zi, 0)
        pltpu.sync_copy(zbuf, s_sh.at[pl.ds(t * ROWS_PER_TILE, ROWS_PER_TILE)])

        _load_node_arrays(as_hbm, ad_hbm, as_loc, ad_loc)
        pltpu.sync_copy(src_hbm.at[pl.ds(wid, 1)], src2d)
        pltpu.sync_copy(dst_hbm.at[pl.ds(wid, 1)], dst2d)
        plsc.subcore_barrier()

        exb = (ex_b0, ex_b1)
        sems = (sem0, sem1)
        sd = [None, None]
        for k in range(chunks):
            jj = k & 1
            if sd[jj] is not None:
                sd[jj].wait()
                sd[jj] = None
            for g in range(GRPS):
                sl = pl.ds(g * LANES, LANES)
                sv = src2d[0, k, sl]
                dv = dst2d[0, k, sl]
                e = (plsc.load_gather(as_loc, [sv])
                     + plsc.load_gather(ad_loc, [dv]))
                e = jnp.maximum(e, 0.2 * e)
                exb[jj][sl] = jnp.exp(e)
            sd[jj] = pltpu.async_copy(
                exb[jj], s_sh.at[dst2d.at[0, k]], sems[jj], add=True)
        for jj in (0, 1):
            if sd[jj] is not None:
                sd[jj].wait()
        plsc.subcore_barrier()
        pltpu.sync_copy(
            s_sh.at[pl.ds(t * ROWS_PER_TILE, ROWS_PER_TILE)],
            s_out.at[c, pl.ds(t * ROWS_PER_TILE, ROWS_PER_TILE)])

    return pass1


def _make_sc_wpass(chunks):
    mesh = plsc.VectorSubcoreMesh(core_axis_name="c", subcore_axis_name="s")
    rec = 3 * B

    @functools.partial(
        pl.kernel,
        out_type=jax.ShapeDtypeStruct((NCORES * NTILES * chunks * rec,), jnp.int32),
        mesh=mesh,
        compiler_params=pltpu.CompilerParams(needs_layout_passes=False),
        scratch_types=[
            pltpu.VMEM((NS,), jnp.float32),
            pltpu.VMEM((NS,), jnp.float32),
            pltpu.VMEM((NS,), jnp.float32),
            pltpu.VMEM((NS,), jnp.float32),
            pltpu.VMEM((1, chunks, B), jnp.int32),
            pltpu.VMEM((1, chunks, B), jnp.int32),
            pltpu.VMEM((chunks * rec,), jnp.int32),
        ],
    )
    def wpass(src_hbm, dst_hbm, as_hbm, ad_hbm, s2_hbm, cmb_out,
              as_loc, ad_loc, s_loc, s_tmp, src2d, dst2d, cmb):
        c = lax.axis_index("c")
        t = lax.axis_index("s")
        wid = c * NTILES + t

        _load_node_arrays(as_hbm, ad_hbm, as_loc, ad_loc)
        pltpu.sync_copy(s2_hbm.at[0], s_loc)
        pltpu.sync_copy(s2_hbm.at[1], s_tmp)

        def si(j, _):
            sl = pl.ds(j * LANES, LANES)
            s_loc[sl] = s_loc[sl] + s_tmp[sl]
            return 0
        lax.fori_loop(0, NS // LANES, si, 0)

        pltpu.sync_copy(src_hbm.at[pl.ds(wid, 1)], src2d)
        pltpu.sync_copy(dst_hbm.at[pl.ds(wid, 1)], dst2d)

        def chunk(k, _):
            for g in range(GRPS):
                sl = pl.ds(g * LANES, LANES)
                sv = src2d[0, k, sl]
                dv = dst2d[0, k, sl]
                e = (plsc.load_gather(as_loc, [sv])
                     + plsc.load_gather(ad_loc, [dv]))
                e = jnp.maximum(e, 0.2 * e)
                ex = jnp.exp(e)
                sg = plsc.load_gather(s_loc, [dv])
                w = ex / sg
                cmb[pl.ds(k * rec + g * LANES, LANES)] = sv
                cmb[pl.ds(k * rec + B + g * LANES, LANES)] = dv
                cmb[pl.ds(k * rec + 2 * B + g * LANES, LANES)] = (
                    plsc.bitcast(w, jnp.int32))
            return 0
        lax.fori_loop(0, chunks, chunk, 0)
        pltpu.sync_copy(cmb, cmb_out.at[pl.ds(wid * chunks * rec, chunks * rec)])

    return wpass


def _make_sc_pass2(ch0, ch1):
    mesh = plsc.VectorSubcoreMesh(core_axis_name="c", subcore_axis_name="s")

    @functools.partial(
        pl.kernel,
        out_type=jax.ShapeDtypeStruct((NCORES, NS, D), jnp.float32),
        mesh=mesh,
        compiler_params=pltpu.CompilerParams(needs_layout_passes=False),
        scratch_types=[
            pltpu.VMEM((3, B), jnp.int32),
            pltpu.VMEM((3, B), jnp.int32),
            pltpu.VMEM((B,), jnp.float32),
            pltpu.VMEM((B, D), jnp.float32),
            pltpu.VMEM((B, D), jnp.float32),
            pltpu.SemaphoreType.DMA,
            pltpu.SemaphoreType.DMA,
            pltpu.SemaphoreType.DMA,
            pltpu.SemaphoreType.DMA,
            pltpu.VMEM_SHARED((NS, D), jnp.float32),
        ],
    )
    def pass2(cmb_hbm, h_hbm, out_hbm,
              cmb0, cmb1, w_b, rows0, rows1,
              semg0, semg1, sems0, sems1, out_sh):
        c = lax.axis_index("c")
        t = lax.axis_index("s")
        my = jnp.where(c == 0, ch0, ch1)
        base = jnp.where(c == 0, t * ch0, NTILES * ch0 + t * ch1)
        bufs = ((cmb0, rows0, semg0, sems0),
                (cmb1, rows1, semg1, sems1))

        def drain(sem, buf):
            pltpu.make_async_copy(h_hbm.at[pl.ds(0, B)], buf, sem).wait()

        def zr(i, _):
            for cc in range(DC):
                rows0[i, pl.ds(cc * LANES, LANES)] = jnp.zeros((LANES,), jnp.float32)
            return 0
        lax.fori_loop(0, B, zr, 0)
        for j in range(WCHUNKS):
            pltpu.sync_copy(rows0, out_sh.at[pl.ds(t * ROWS_PER_TILE + j * B, B)])
        plsc.subcore_barrier()

        pltpu.sync_copy(cmb_hbm.at[base], cmb0)
        pltpu.async_copy(h_hbm.at[cmb0.at[0]], rows0, semg0)

        def pair(p, _):
            for jj in (0, 1):
                cmb, rows, semg, sems = bufs[jj]
                ncmb, nrows, nsemg, nsems = bufs[jj ^ 1]
                k = 2 * p + jj

                if jj == 0:
                    @pl.when(p >= 1)
                    def _():
                        drain(nsems, nrows)
                else:
                    drain(nsems, nrows)

                if jj == 0:
                    pltpu.sync_copy(cmb_hbm.at[base + k + 1], ncmb)
                    pltpu.async_copy(h_hbm.at[ncmb.at[0]], nrows, nsemg)
                else:
                    @pl.when(p + 1 < my // 2)
                    def _():
                        pltpu.sync_copy(cmb_hbm.at[base + k + 1], ncmb)
                        pltpu.async_copy(h_hbm.at[ncmb.at[0]], nrows, nsemg)

                drain(semg, rows)

                for g in range(GRPS):
                    sl = pl.ds(g * LANES, LANES)
                    w_b[sl] = plsc.bitcast(cmb[2, sl], jnp.float32)

                def scale(i, _):
                    for d in range(4):
                        r = 4 * i + d
                        bvec = jnp.full((LANES,), r, jnp.int32)
                        w = plsc.load_gather(w_b, [bvec])
                        for cc in range(DC):
                            sl2 = pl.ds(cc * LANES, LANES)
                            rows[r, sl2] = rows[r, sl2] * w
                    return 0
                lax.fori_loop(0, B // 4, scale, 0)

                pltpu.async_copy(rows, out_sh.at[cmb.at[1]], sems, add=True)
            return 0
        lax.fori_loop(0, my // 2, pair, 0)
        drain(sems1, rows1)
        plsc.subcore_barrier()
        for j in range(WCHUNKS):
            off = t * ROWS_PER_TILE + j * B
            pltpu.sync_copy(out_sh.at[pl.ds(off, B)],
                            out_hbm.at[c, pl.ds(off, B)])

    return pass2


def _final_body(p_ref, b1_ref, xg_ref, am_ref, wfc_ref, bfc_ref, out_ref):
    p = p_ref[...]
    nf = p[0, 0] + p[1, 0] + b1_ref[...][None, :]
    xg = xg_ref[...][0]
    am = am_ref[...][0]
    impg = jnp.sum(nf * xg, axis=1, keepdims=True)
    impa = jnp.sum(nf * am, axis=1, keepdims=True)
    upd = nf + impg * xg + impa * am
    out = jnp.dot(upd, wfc_ref[...], preferred_element_type=jnp.float32)
    out_ref[...] = (out + bfc_ref[...][None, :])[None]


def _final(parts, b1, xg, am, wfc, bfc):
    return pl.pallas_call(
        _final_body,
        grid=(G,),
        in_specs=[
            pl.BlockSpec((2, 1, P, D), lambda g: (0, g, 0, 0)),
            pl.BlockSpec((D,), lambda g: (0,)),
            pl.BlockSpec((1, 1, D), lambda g: (g, 0, 0)),
            pl.BlockSpec((1, 1, D), lambda g: (g, 0, 0)),
            pl.BlockSpec((D, D), lambda g: (0, 0)),
            pl.BlockSpec((D,), lambda g: (0,)),
        ],
        out_specs=pl.BlockSpec((1, P, D), lambda g: (g, 0, 0)),
        out_shape=jax.ShapeDtypeStruct((G, P, D), jnp.float32),
    )(parts, b1, xg.reshape(G, 1, D), am.reshape(G, 1, D), wfc, bfc)


def kernel(x1, edge_index1, x2, edge_index2, group_index,
           W1, a1_src, a1_dst, b1, W2, a2_src, a2_dst, b2, Wfc, bfc):
    e = edge_index1.shape[1]
    e_total = e + N
    ntile_all = NCORES * NTILES
    chunks = -(-e_total // (ntile_all * B))
    chunks += chunks % 2
    epad = ntile_all * B * chunks
    npad = epad - e_total

    loop = jnp.arange(N, dtype=jnp.int32)
    src_all = jnp.concatenate([
        edge_index1[0].astype(jnp.int32), loop,
        jnp.zeros((npad,), jnp.int32)]).reshape(ntile_all, chunks, B)
    dst_all = jnp.concatenate([
        edge_index1[1].astype(jnp.int32), loop,
        TRASH + (jnp.arange(npad, dtype=jnp.int32) % (NS - N))]).reshape(
            ntile_all, chunks, B)

    h, as1, ad1 = _project(x1, W1, a1_src, a1_dst)
    as1 = as1.reshape(N)
    ad1 = ad1.reshape(N)
    s2 = _make_sc_pass1(chunks)(src_all, dst_all, as1, ad1)
    cmb = _make_sc_wpass(chunks)(src_all, dst_all, as1, ad1, s2)
    cmb = cmb.reshape(ntile_all * chunks, 3, B)
    parts = _make_sc_pass2(chunks + 58, chunks - 58)(cmb, h)
    xg, am = _group_gat(x2, W2, a2_src, a2_dst, b2, edge_index2)

    parts4 = parts[:, :N, :].reshape(NCORES, G, P, D)
    out1 = _final(parts4, b1, xg, am, Wfc, bfc).reshape(N, D)
    return out1, xg


# speedup vs baseline: 1.0136x; 1.0136x over previous
import functools

import jax
import jax.numpy as jnp
from jax import lax
from jax.experimental import pallas as pl
from jax.experimental.pallas import tpu as pltpu
from jax.experimental.pallas import tpu_sc as plsc

N = 10000
D = 128
G = 100
P = 100
LANES = 16
NTILES = 16
NCORES = 2
NS = 10240
TRASH = N
B = 128
ROWS_PER_TILE = NS // NTILES
ZCHUNKS = ROWS_PER_TILE // LANES
WCHUNKS = ROWS_PER_TILE // B
GRPS = B // LANES
DC = D // LANES


def _proj_body(x_ref, w_ref, asrc_ref, adst_ref, h_ref, as_ref, ad_ref):
    h = jnp.dot(x_ref[...], w_ref[...], preferred_element_type=jnp.float32)
    h_ref[...] = h
    as_ref[...] = jnp.sum(h * asrc_ref[...], axis=1, keepdims=True)
    ad_ref[...] = jnp.sum(h * adst_ref[...], axis=1, keepdims=True)


def _project(x, w, a_src, a_dst):
    n = x.shape[0]
    blk = 2000
    grid = n // blk
    return pl.pallas_call(
        _proj_body,
        grid=(grid,),
        in_specs=[
            pl.BlockSpec((blk, D), lambda i: (i, 0)),
            pl.BlockSpec((D, D), lambda i: (0, 0)),
            pl.BlockSpec((1, D), lambda i: (0, 0)),
            pl.BlockSpec((1, D), lambda i: (0, 0)),
        ],
        out_specs=[
            pl.BlockSpec((blk, D), lambda i: (i, 0)),
            pl.BlockSpec((blk, 1), lambda i: (i, 0)),
            pl.BlockSpec((blk, 1), lambda i: (i, 0)),
        ],
        out_shape=[
            jax.ShapeDtypeStruct((n, D), jnp.float32),
            jax.ShapeDtypeStruct((n, 1), jnp.float32),
            jax.ShapeDtypeStruct((n, 1), jnp.float32),
        ],
    )(x, w, a_src, a_dst)


def _group_body(x2_ref, w2_ref, a2s_ref, a2d_ref, b2_ref, ei2_ref,
                xg_ref, am_ref):
    h2 = jnp.dot(x2_ref[...], w2_ref[...], preferred_element_type=jnp.float32)
    as2 = jnp.sum(h2 * a2s_ref[...], axis=1)
    ad2 = jnp.sum(h2 * a2d_ref[...], axis=1)
    ei = ei2_ref[...]
    eg = ei.shape[1]
    gi = lax.broadcasted_iota(jnp.int32, (eg, G), 1)
    src_oh = (ei[0][:, None] == gi).astype(jnp.float32)
    dst_oh = (ei[1][:, None] == gi).astype(jnp.float32)
    m_cnt = lax.dot_general(src_oh, dst_oh, (((0,), (0,)), ((), ())),
                            preferred_element_type=jnp.float32)
    eye = (lax.broadcasted_iota(jnp.int32, (G, G), 0)
           == lax.broadcasted_iota(jnp.int32, (G, G), 1)).astype(jnp.float32)
    mp = m_cnt + eye
    emat = as2[:, None] + ad2[None, :]
    el = jnp.maximum(emat, 0.2 * emat)
    mx = jnp.max(jnp.where(mp > 0, el, -1e30), axis=0)
    z = mp * jnp.exp(el - mx[None, :])
    ssum = jnp.sum(z, axis=0)
    num = lax.dot_general(z, h2, (((0,), (0,)), ((), ())),
                          preferred_element_type=jnp.float32)
    xg = num / (ssum[:, None] + 1e-16) + b2_ref[...][None, :]
    xg_ref[...] = xg
    cnt = jnp.sum(m_cnt, axis=1)
    adj_sum = jnp.dot(m_cnt, xg, preferred_element_type=jnp.float32)
    am_ref[...] = adj_sum / jnp.maximum(cnt, 1.0)[:, None]


def _group_gat(x2, w2, a2_src, a2_dst, b2, ei2):
    return pl.pallas_call(
        _group_body,
        out_shape=[
            jax.ShapeDtypeStruct((G, D), jnp.float32),
            jax.ShapeDtypeStruct((G, D), jnp.float32),
        ],
    )(x2, w2, a2_src, a2_dst, b2, ei2)


def _load_node_arrays(as_hbm, ad_hbm, as_loc, ad_loc):
    pltpu.sync_copy(as_hbm, as_loc.at[pl.ds(0, N)])
    pltpu.sync_copy(ad_hbm, ad_loc.at[pl.ds(0, N)])
    for i in range((NS - N) // LANES):
        as_loc[pl.ds(N + i * LANES, LANES)] = jnp.zeros((LANES,), jnp.float32)
        ad_loc[pl.ds(N + i * LANES, LANES)] = jnp.zeros((LANES,), jnp.float32)


def _make_sc_pass1(chunks):
    mesh = plsc.VectorSubcoreMesh(core_axis_name="c", subcore_axis_name="s")

    @functools.partial(
        pl.kernel,
        out_type=jax.ShapeDtypeStruct((NCORES, NS), jnp.float32),
        mesh=mesh,
        compiler_params=pltpu.CompilerParams(needs_layout_passes=False),
        scratch_types=[
            pltpu.VMEM((NS,), jnp.float32),
            pltpu.VMEM((NS,), jnp.float32),
            pltpu.VMEM((1, chunks, B), jnp.int32),
            pltpu.VMEM((1, chunks, B), jnp.int32),
            pltpu.VMEM((B,), jnp.float32),
            pltpu.VMEM((B,), jnp.float32),
            pltpu.SemaphoreType.DMA,
            pltpu.SemaphoreType.DMA,
            pltpu.VMEM((ROWS_PER_TILE,), jnp.float32),
            pltpu.VMEM_SHARED((NS,), jnp.float32),
        ],
    )
    def pass1(src_hbm, dst_hbm, as_hbm, ad_hbm, s_out,
              as_loc, ad_loc, src2d, dst2d, ex_b0, ex_b1, sem0, sem1,
              zbuf, s_sh):
        c = lax.axis_index("c")
        t = lax.axis_index("s")
        wid = c * NTILES + t

        def zi(i, _):
            zbuf[pl.ds(i * LANES, LANES)] = jnp.zeros((LANES,), jnp.float32)
            return 0
        lax.fori_loop(0, ZCHUNKS, zi, 0)
        pltpu.sync_copy(zbuf, s_sh.at[pl.ds(t * ROWS_PER_TILE, ROWS_PER_TILE)])

        _load_node_arrays(as_hbm, ad_hbm, as_loc, ad_loc)
        pltpu.sync_copy(src_hbm.at[pl.ds(wid, 1)], src2d)
        pltpu.sync_copy(dst_hbm.at[pl.ds(wid, 1)], dst2d)
        plsc.subcore_barrier()

        exb = (ex_b0, ex_b1)
        sems = (sem0, sem1)
        sd = [None, None]
        for k in range(chunks):
            jj = k & 1
            if sd[jj] is not None:
                sd[jj].wait()
                sd[jj] = None
            for g in range(GRPS):
                sl = pl.ds(g * LANES, LANES)
                sv = src2d[0, k, sl]
                dv = dst2d[0, k, sl]
                e = (plsc.load_gather(as_loc, [sv])
                     + plsc.load_gather(ad_loc, [dv]))
                e = jnp.maximum(e, 0.2 * e)
                exb[jj][sl] = jnp.exp(e)
            sd[jj] = pltpu.async_copy(
                exb[jj], s_sh.at[dst2d.at[0, k]], sems[jj], add=True)
        for jj in (0, 1):
            if sd[jj] is not None:
                sd[jj].wait()
        plsc.subcore_barrier()
        pltpu.sync_copy(
            s_sh.at[pl.ds(t * ROWS_PER_TILE, ROWS_PER_TILE)],
            s_out.at[c, pl.ds(t * ROWS_PER_TILE, ROWS_PER_TILE)])

    return pass1


def _make_sc_wpass(chunks):
    mesh = plsc.VectorSubcoreMesh(core_axis_name="c", subcore_axis_name="s")
    rec = 3 * B

    @functools.partial(
        pl.kernel,
        out_type=jax.ShapeDtypeStruct((NCORES * NTILES * chunks * rec,), jnp.int32),
        mesh=mesh,
        compiler_params=pltpu.CompilerParams(needs_layout_passes=False),
        scratch_types=[
            pltpu.VMEM((NS,), jnp.float32),
            pltpu.VMEM((NS,), jnp.float32),
            pltpu.VMEM((NS,), jnp.float32),
            pltpu.VMEM((NS,), jnp.float32),
            pltpu.VMEM((1, chunks, B), jnp.int32),
            pltpu.VMEM((1, chunks, B), jnp.int32),
            pltpu.VMEM((chunks * rec,), jnp.int32),
        ],
    )
    def wpass(src_hbm, dst_hbm, as_hbm, ad_hbm, s2_hbm, cmb_out,
              as_loc, ad_loc, s_loc, s_tmp, src2d, dst2d, cmb):
        c = lax.axis_index("c")
        t = lax.axis_index("s")
        wid = c * NTILES + t

        _load_node_arrays(as_hbm, ad_hbm, as_loc, ad_loc)
        pltpu.sync_copy(s2_hbm.at[0], s_loc)
        pltpu.sync_copy(s2_hbm.at[1], s_tmp)

        def si(j, _):
            sl = pl.ds(j * LANES, LANES)
            s_loc[sl] = s_loc[sl] + s_tmp[sl]
            return 0
        lax.fori_loop(0, NS // LANES, si, 0)

        pltpu.sync_copy(src_hbm.at[pl.ds(wid, 1)], src2d)
        pltpu.sync_copy(dst_hbm.at[pl.ds(wid, 1)], dst2d)

        def chunk(k, _):
            for g in range(GRPS):
                sl = pl.ds(g * LANES, LANES)
                sv = src2d[0, k, sl]
                dv = dst2d[0, k, sl]
                e = (plsc.load_gather(as_loc, [sv])
                     + plsc.load_gather(ad_loc, [dv]))
                e = jnp.maximum(e, 0.2 * e)
                ex = jnp.exp(e)
                sg = plsc.load_gather(s_loc, [dv])
                w = ex / sg
                cmb[pl.ds(k * rec + g * LANES, LANES)] = sv
                cmb[pl.ds(k * rec + B + g * LANES, LANES)] = dv
                cmb[pl.ds(k * rec + 2 * B + g * LANES, LANES)] = (
                    plsc.bitcast(w, jnp.int32))
            return 0
        lax.fori_loop(0, chunks, chunk, 0)
        pltpu.sync_copy(cmb, cmb_out.at[pl.ds(wid * chunks * rec, chunks * rec)])

    return wpass


def _make_sc_pass2(ch0, ch1):
    mesh = plsc.VectorSubcoreMesh(core_axis_name="c", subcore_axis_name="s")

    @functools.partial(
        pl.kernel,
        out_type=jax.ShapeDtypeStruct((NCORES, NS, D), jnp.float32),
        mesh=mesh,
        compiler_params=pltpu.CompilerParams(needs_layout_passes=False),
        scratch_types=[
            pltpu.VMEM((3, B), jnp.int32),
            pltpu.VMEM((3, B), jnp.int32),
            pltpu.VMEM((B,), jnp.float32),
            pltpu.VMEM((B, D), jnp.float32),
            pltpu.VMEM((B, D), jnp.float32),
            pltpu.SemaphoreType.DMA,
            pltpu.SemaphoreType.DMA,
            pltpu.SemaphoreType.DMA,
            pltpu.SemaphoreType.DMA,
            pltpu.VMEM_SHARED((NS, D), jnp.float32),
        ],
    )
    def pass2(cmb_hbm, h_hbm, out_hbm,
              cmb0, cmb1, w_b, rows0, rows1,
              semg0, semg1, sems0, sems1, out_sh):
        c = lax.axis_index("c")
        t = lax.axis_index("s")
        my = jnp.where(c == 0, ch0, ch1)
        base = jnp.where(c == 0, t * ch0, NTILES * ch0 + t * ch1)
        bufs = ((cmb0, rows0, semg0, sems0),
                (cmb1, rows1, semg1, sems1))

        def drain(sem, buf):
            pltpu.make_async_copy(h_hbm.at[pl.ds(0, B)], buf, sem).wait()

        def zr(i, _):
            for cc in range(DC):
                rows0[i, pl.ds(cc * LANES, LANES)] = jnp.zeros((LANES,), jnp.float32)
            return 0
        lax.fori_loop(0, B, zr, 0)
        zds = [pltpu.async_copy(
            rows0, out_sh.at[pl.ds(t * ROWS_PER_TILE + j * B, B)], semg0)
            for j in range(WCHUNKS)]
        for zd in zds:
            zd.wait()
        plsc.subcore_barrier()

        pltpu.sync_copy(cmb_hbm.at[base], cmb0)
        pltpu.async_copy(h_hbm.at[cmb0.at[0]], rows0, semg0)

        def pair(p, _):
            for jj in (0, 1):
                cmb, rows, semg, sems = bufs[jj]
                ncmb, nrows, nsemg, nsems = bufs[jj ^ 1]
                k = 2 * p + jj

                if jj == 0:
                    @pl.when(p >= 1)
                    def _():
                        drain(nsems, nrows)
                else:
                    drain(nsems, nrows)

                if jj == 0:
                    pltpu.sync_copy(cmb_hbm.at[base + k + 1], ncmb)
                    pltpu.async_copy(h_hbm.at[ncmb.at[0]], nrows, nsemg)
                else:
                    @pl.when(p + 1 < my // 2)
                    def _():
                        pltpu.sync_copy(cmb_hbm.at[base + k + 1], ncmb)
                        pltpu.async_copy(h_hbm.at[ncmb.at[0]], nrows, nsemg)

                drain(semg, rows)

                for g in range(GRPS):
                    sl = pl.ds(g * LANES, LANES)
                    w_b[sl] = plsc.bitcast(cmb[2, sl], jnp.float32)

                def scale(i, _):
                    for d in range(4):
                        r = 4 * i + d
                        bvec = jnp.full((LANES,), r, jnp.int32)
                        w = plsc.load_gather(w_b, [bvec])
                        for cc in range(DC):
                            sl2 = pl.ds(cc * LANES, LANES)
                            rows[r, sl2] = rows[r, sl2] * w
                    return 0
                lax.fori_loop(0, B // 4, scale, 0)

                pltpu.async_copy(rows, out_sh.at[cmb.at[1]], sems, add=True)
            return 0
        lax.fori_loop(0, my // 2, pair, 0)
        drain(sems1, rows1)
        plsc.subcore_barrier()
        wds = []
        for j in range(WCHUNKS):
            off = t * ROWS_PER_TILE + j * B
            wds.append(pltpu.async_copy(
                out_sh.at[pl.ds(off, B)], out_hbm.at[c, pl.ds(off, B)],
                (semg0, semg1, sems0, sems1, semg0)[j]))
        for wd in wds:
            wd.wait()

    return pass2


def _final_body(p_ref, b1_ref, xg_ref, am_ref, wfc_ref, bfc_ref, out_ref):
    p = p_ref[...]
    nf = p[0, 0] + p[1, 0] + b1_ref[...][None, :]
    xg = xg_ref[...][0]
    am = am_ref[...][0]
    impg = jnp.sum(nf * xg, axis=1, keepdims=True)
    impa = jnp.sum(nf * am, axis=1, keepdims=True)
    upd = nf + impg * xg + impa * am
    out = jnp.dot(upd, wfc_ref[...], preferred_element_type=jnp.float32)
    out_ref[...] = (out + bfc_ref[...][None, :])[None]


def _final(parts, b1, xg, am, wfc, bfc):
    return pl.pallas_call(
        _final_body,
        grid=(G,),
        in_specs=[
            pl.BlockSpec((2, 1, P, D), lambda g: (0, g, 0, 0)),
            pl.BlockSpec((D,), lambda g: (0,)),
            pl.BlockSpec((1, 1, D), lambda g: (g, 0, 0)),
            pl.BlockSpec((1, 1, D), lambda g: (g, 0, 0)),
            pl.BlockSpec((D, D), lambda g: (0, 0)),
            pl.BlockSpec((D,), lambda g: (0,)),
        ],
        out_specs=pl.BlockSpec((1, P, D), lambda g: (g, 0, 0)),
        out_shape=jax.ShapeDtypeStruct((G, P, D), jnp.float32),
    )(parts, b1, xg.reshape(G, 1, D), am.reshape(G, 1, D), wfc, bfc)


def kernel(x1, edge_index1, x2, edge_index2, group_index,
           W1, a1_src, a1_dst, b1, W2, a2_src, a2_dst, b2, Wfc, bfc):
    e = edge_index1.shape[1]
    e_total = e + N
    ntile_all = NCORES * NTILES
    chunks = -(-e_total // (ntile_all * B))
    chunks += chunks % 2
    epad = ntile_all * B * chunks
    npad = epad - e_total

    loop = jnp.arange(N, dtype=jnp.int32)
    src_all = jnp.concatenate([
        edge_index1[0].astype(jnp.int32), loop,
        jnp.zeros((npad,), jnp.int32)]).reshape(ntile_all, chunks, B)
    dst_all = jnp.concatenate([
        edge_index1[1].astype(jnp.int32), loop,
        TRASH + (jnp.arange(npad, dtype=jnp.int32) % (NS - N))]).reshape(
            ntile_all, chunks, B)

    h, as1, ad1 = _project(x1, W1, a1_src, a1_dst)
    as1 = as1.reshape(N)
    ad1 = ad1.reshape(N)
    s2 = _make_sc_pass1(chunks)(src_all, dst_all, as1, ad1)
    cmb = _make_sc_wpass(chunks)(src_all, dst_all, as1, ad1, s2)
    cmb = cmb.reshape(ntile_all * chunks, 3, B)
    parts = _make_sc_pass2(chunks + 36, chunks - 36)(cmb, h)
    xg, am = _group_gat(x2, W2, a2_src, a2_dst, b2, edge_index2)

    parts4 = parts[:, :N, :].reshape(NCORES, G, P, D)
    out1 = _final(parts4, b1, xg, am, Wfc, bfc).reshape(N, D)
    return out1, xg
